# baseline (speedup 1.0000x reference)
import functools

import jax
import jax.numpy as jnp
from jax.experimental import pallas as pl
from jax.experimental.pallas import tpu as pltpu

N_ATOMS = 256
N_HKL = 50000
N_BINS = 20
CELL = 50.0

NP = 51200
NB = 16
R = NP // NB
BINS_PAD = 32


def _log_i0(x):
    ax = jnp.abs(x)
    t = jnp.square(jnp.minimum(ax, 3.75) / 3.75)
    p_small = 1.0 + t * (3.5156229 + t * (3.0899424 + t * (1.2067492
        + t * (0.2659732 + t * (0.0360768 + t * 0.0045813)))))
    small = jnp.log(p_small)
    xl = jnp.maximum(ax, 3.75)
    u = 3.75 / xl
    p_large = 0.39894228 + u * (0.01328592 + u * (0.00225319 + u * (-0.00157565
        + u * (0.00916281 + u * (-0.02057706 + u * (0.02635537
        + u * (-0.01647633 + u * 0.00392377)))))))
    large = xl - 0.5 * jnp.log(xl) + jnp.log(p_large)
    return jnp.where(ax <= 3.75, small, large)


def _sincos_2pi(d):
    u = d - jnp.round(d)
    x = jnp.float32(2.0 * jnp.pi) * u
    z = x * x
    c = 1.0 + z * (-0.5 + z * (1.0 / 24 + z * (-1.0 / 720 + z * (
        1.0 / 40320 + z * (-1.0 / 3628800 + z * (1.0 / 479001600 + z * (
            -1.0 / 87178291200)))))))
    s = x * (1.0 + z * (-1.0 / 6 + z * (1.0 / 120 + z * (-1.0 / 5040 + z * (
        1.0 / 362880 + z * (-1.0 / 39916800 + z * (1.0 / 6227020800)))))))
    return c, s


def _llg_kernel(frac_ref, hklt_ref, eobs_ref, eps_ref, dobs_ref, cent_ref,
                lab_ref, sub_ref, valid_ref, sa_ref, out_ref, fm_scr):
    frac = frac_ref[...]

    bins2d = jax.lax.broadcasted_iota(jnp.int32, (BINS_PAD, R), 0)

    seg = jnp.zeros((BINS_PAD, R), jnp.float32)
    cnt = jnp.zeros((BINS_PAD, R), jnp.float32)

    for i in range(NB):
        hkl_blk = hklt_ref[:, i * R:(i + 1) * R]
        d = jax.lax.dot_general(
            frac, hkl_blk, (((1,), (0,)), ((), ())),
            preferred_element_type=jnp.float32)
        c, s = _sincos_2pi(d)
        fc_re = jnp.sum(c, axis=0, keepdims=True)
        fc_im = jnp.sum(s, axis=0, keepdims=True)
        fm = jnp.sqrt(fc_re * fc_re + fc_im * fc_im + 1e-12)
        fm_scr[i:i + 1, :] = fm

        valid_row = valid_ref[i:i + 1, :]
        vals = fm * fm / eps_ref[i:i + 1, :] * valid_row
        m = bins2d == lab_ref[i:i + 1, :]
        seg = seg + jnp.where(m, vals, 0.0)
        cnt = cnt + jnp.where(m, valid_row, 0.0)

    seg_t = jnp.sum(seg, axis=1, keepdims=True)
    cnt_t = jnp.sum(cnt, axis=1, keepdims=True)
    sigma_p = seg_t / jnp.maximum(cnt_t, 1.0)
    sa_clip = jnp.clip(sa_ref[...], 0.015, 0.99)

    labels = lab_ref[...]
    sp_g = jnp.zeros((NB, R), jnp.float32)
    sa_g = jnp.zeros((NB, R), jnp.float32)
    for b in range(N_BINS):
        mb = labels == b
        sp_g = jnp.where(mb, sigma_p[b, 0], sp_g)
        sa_g = jnp.where(mb, sa_clip[b, 0], sa_g)

    fm_all = fm_scr[...]
    eobs = eobs_ref[...]
    dobs = dobs_ref[...]
    eps = eps_ref[...]

    ecalc = fm_all / jnp.sqrt(eps * sp_g)
    dsa = dobs * sa_g
    t = 1.0 - dsa * dsa
    e2 = eobs * eobs + ecalc * ecalc
    arg_a = 2.0 * dsa * eobs * ecalc / t
    llg_a = -jnp.log(t) - dsa * dsa * e2 / t + _log_i0(arg_a)
    arg_c = jnp.abs(0.5 * arg_a)
    logcosh = arg_c + jnp.log(1.0 + jnp.exp(-2.0 * arg_c)) - jnp.float32(jnp.log(2.0))
    llg_c = -0.5 * jnp.log(t) - dsa * dsa * e2 / (2.0 * t) + logcosh
    llg = jnp.where(cent_ref[...] != 0.0, llg_c, llg_a)
    out_ref[...] = jnp.sum(llg * sub_ref[...]).reshape(1, 1)


@functools.partial(jax.jit, static_argnames=())
def kernel(xyz_ort, Eobs, Eps, Dobs, sigmaAs, Centric, bin_labels, hkl, sub_mask):
    f32 = jnp.float32
    frac = (xyz_ort / CELL).astype(f32)

    def pad1(x, fill):
        return jnp.pad(x.astype(f32), (0, NP - N_HKL), constant_values=fill)

    hkl_t = jnp.pad(hkl.astype(f32).T, ((0, 0), (0, NP - N_HKL)))
    eobs = pad1(Eobs, 0.0).reshape(NB, R)
    eps = pad1(Eps, 1.0).reshape(NB, R)
    dobs = pad1(Dobs, 0.0).reshape(NB, R)
    cent = pad1(Centric.astype(f32), 0.0).reshape(NB, R)
    sub = pad1(sub_mask.astype(f32), 0.0).reshape(NB, R)
    valid = pad1(jnp.ones((N_HKL,), f32), 0.0).reshape(NB, R)
    labels = jnp.pad(bin_labels, (0, NP - N_HKL)).reshape(NB, R)
    sa = jnp.pad(sigmaAs.astype(f32), (0, BINS_PAD - N_BINS)).reshape(BINS_PAD, 1)

    out = pl.pallas_call(
        _llg_kernel,
        out_shape=jax.ShapeDtypeStruct((1, 1), f32),
        scratch_shapes=[
            pltpu.VMEM((NB, R), f32),
        ],
    )(frac, hkl_t, eobs, eps, dobs, cent, labels, sub, valid, sa)
    return out[0, 0]


# speedup vs baseline: 1.0386x; 1.0386x over previous
import functools

import jax
import jax.numpy as jnp
from jax.experimental import pallas as pl
from jax.experimental.pallas import tpu as pltpu

N_ATOMS = 256
N_HKL = 50000
N_BINS = 20
CELL = 50.0

NP = 51200
NB = 16
R = NP // NB
BINS_PAD = 32


def _log_i0(x):
    ax = jnp.abs(x)
    t = jnp.square(jnp.minimum(ax, 3.75) / 3.75)
    p_small = 1.0 + t * (3.5156229 + t * (3.0899424 + t * (1.2067492
        + t * (0.2659732 + t * (0.0360768 + t * 0.0045813)))))
    small = jnp.log(p_small)
    xl = jnp.maximum(ax, 3.75)
    u = 3.75 / xl
    p_large = 0.39894228 + u * (0.01328592 + u * (0.00225319 + u * (-0.00157565
        + u * (0.00916281 + u * (-0.02057706 + u * (0.02635537
        + u * (-0.01647633 + u * 0.00392377)))))))
    large = xl - 0.5 * jnp.log(xl) + jnp.log(p_large)
    return jnp.where(ax <= 3.75, small, large)


def _sincos_2pi(d):
    u = d - jnp.round(d)
    a = jnp.abs(u)
    b = jnp.minimum(a, 0.5 - a)
    x = jnp.float32(2.0 * jnp.pi) * b
    z = x * x
    cosp = 1.0 + z * (-0.5 + z * (1.0 / 24 + z * (-1.0 / 720 + z * (1.0 / 40320))))
    sinp = x * (1.0 + z * (-1.0 / 6 + z * (1.0 / 120 + z * (-1.0 / 5040 + z * (
        1.0 / 362880)))))
    c = jnp.where(a > 0.25, -cosp, cosp)
    s = jnp.where(u < 0.0, -sinp, sinp)
    return c, s


def _llg_kernel(frac_ref, hklt_ref, eobs_ref, eps_ref, dobs_ref, cent_ref,
                lab_ref, sub_ref, valid_ref, sa_ref, out_ref, fm_scr):
    frac = frac_ref[...]

    bins2d = jax.lax.broadcasted_iota(jnp.int32, (BINS_PAD, R), 0)

    seg = jnp.zeros((BINS_PAD, R), jnp.float32)
    cnt = jnp.zeros((BINS_PAD, R), jnp.float32)

    for i in range(NB):
        hkl_blk = hklt_ref[:, i * R:(i + 1) * R]
        d = jax.lax.dot_general(
            frac, hkl_blk, (((1,), (0,)), ((), ())),
            preferred_element_type=jnp.float32)
        c, s = _sincos_2pi(d)
        fc_re = jnp.sum(c, axis=0, keepdims=True)
        fc_im = jnp.sum(s, axis=0, keepdims=True)
        fm = jnp.sqrt(fc_re * fc_re + fc_im * fc_im + 1e-12)
        fm_scr[i:i + 1, :] = fm

        valid_row = valid_ref[i:i + 1, :]
        vals = fm * fm / eps_ref[i:i + 1, :] * valid_row
        m = bins2d == lab_ref[i:i + 1, :]
        seg = seg + jnp.where(m, vals, 0.0)
        cnt = cnt + jnp.where(m, valid_row, 0.0)

    seg_t = jnp.sum(seg, axis=1, keepdims=True)
    cnt_t = jnp.sum(cnt, axis=1, keepdims=True)
    sigma_p = seg_t / jnp.maximum(cnt_t, 1.0)
    sa_clip = jnp.clip(sa_ref[...], 0.015, 0.99)

    labels = lab_ref[...]
    sp_g = jnp.zeros((NB, R), jnp.float32)
    sa_g = jnp.zeros((NB, R), jnp.float32)
    for b in range(N_BINS):
        mb = labels == b
        sp_g = jnp.where(mb, sigma_p[b, 0], sp_g)
        sa_g = jnp.where(mb, sa_clip[b, 0], sa_g)

    fm_all = fm_scr[...]
    eobs = eobs_ref[...]
    dobs = dobs_ref[...]
    eps = eps_ref[...]

    ecalc = fm_all / jnp.sqrt(eps * sp_g)
    dsa = dobs * sa_g
    t = 1.0 - dsa * dsa
    e2 = eobs * eobs + ecalc * ecalc
    arg_a = 2.0 * dsa * eobs * ecalc / t
    llg_a = -jnp.log(t) - dsa * dsa * e2 / t + _log_i0(arg_a)
    arg_c = jnp.abs(0.5 * arg_a)
    logcosh = arg_c + jnp.log(1.0 + jnp.exp(-2.0 * arg_c)) - jnp.float32(jnp.log(2.0))
    llg_c = -0.5 * jnp.log(t) - dsa * dsa * e2 / (2.0 * t) + logcosh
    llg = jnp.where(cent_ref[...] != 0.0, llg_c, llg_a)
    out_ref[...] = jnp.sum(llg * sub_ref[...]).reshape(1, 1)


@functools.partial(jax.jit, static_argnames=())
def kernel(xyz_ort, Eobs, Eps, Dobs, sigmaAs, Centric, bin_labels, hkl, sub_mask):
    f32 = jnp.float32
    frac = (xyz_ort / CELL).astype(f32)

    def pad1(x, fill):
        return jnp.pad(x.astype(f32), (0, NP - N_HKL), constant_values=fill)

    hkl_t = jnp.pad(hkl.astype(f32).T, ((0, 0), (0, NP - N_HKL)))
    eobs = pad1(Eobs, 0.0).reshape(NB, R)
    eps = pad1(Eps, 1.0).reshape(NB, R)
    dobs = pad1(Dobs, 0.0).reshape(NB, R)
    cent = pad1(Centric.astype(f32), 0.0).reshape(NB, R)
    sub = pad1(sub_mask.astype(f32), 0.0).reshape(NB, R)
    valid = pad1(jnp.ones((N_HKL,), f32), 0.0).reshape(NB, R)
    labels = jnp.pad(bin_labels, (0, NP - N_HKL)).reshape(NB, R)
    sa = jnp.pad(sigmaAs.astype(f32), (0, BINS_PAD - N_BINS)).reshape(BINS_PAD, 1)

    out = pl.pallas_call(
        _llg_kernel,
        out_shape=jax.ShapeDtypeStruct((1, 1), f32),
        scratch_shapes=[
            pltpu.VMEM((NB, R), f32),
        ],
    )(frac, hkl_t, eobs, eps, dobs, cent, labels, sub, valid, sa)
    return out[0, 0]


# speedup vs baseline: 1.3531x; 1.3029x over previous
import functools

import jax
import jax.numpy as jnp
from jax.experimental import pallas as pl
from jax.experimental.pallas import tpu as pltpu

N_ATOMS = 256
N_HKL = 50000
N_BINS = 20
CELL = 50.0

NP = 51200
NB = 16
R = NP // NB
BINS_PAD = 32


def _log_i0(x):
    ax = jnp.abs(x)
    t = jnp.square(jnp.minimum(ax, 3.75) / 3.75)
    p_small = 1.0 + t * (3.5156229 + t * (3.0899424 + t * (1.2067492
        + t * (0.2659732 + t * (0.0360768 + t * 0.0045813)))))
    small = jnp.log(p_small)
    xl = jnp.maximum(ax, 3.75)
    u = 3.75 / xl
    p_large = 0.39894228 + u * (0.01328592 + u * (0.00225319 + u * (-0.00157565
        + u * (0.00916281 + u * (-0.02057706 + u * (0.02635537
        + u * (-0.01647633 + u * 0.00392377)))))))
    large = xl - 0.5 * jnp.log(xl) + jnp.log(p_large)
    return jnp.where(ax <= 3.75, small, large)


def _sincos_2pi(d):
    u = d - jnp.round(d)
    z = u * u
    c = 0.9999814 + z * (-19.73259 + z * (64.69856 + z * (
        -82.54686 + z * 45.912495)))
    s = u * (6.2831745 + z * (-41.337803 + z * (81.46372 + z * (
        -75.001564 + z * 33.720444))))
    return c, s


def _llg_kernel(frac_ref, hklt_ref, eobs_ref, eps_ref, dobs_ref, cent_ref,
                lab_ref, sub_ref, valid_ref, sa_ref, out_ref, fm_scr):
    frac = frac_ref[...]

    bins2d = jax.lax.broadcasted_iota(jnp.int32, (BINS_PAD, R), 0)

    seg = jnp.zeros((BINS_PAD, R), jnp.float32)
    cnt = jnp.zeros((BINS_PAD, R), jnp.float32)

    for i in range(NB):
        hkl_blk = hklt_ref[:, i * R:(i + 1) * R]
        d = jax.lax.dot_general(
            frac, hkl_blk, (((1,), (0,)), ((), ())),
            preferred_element_type=jnp.float32)
        c, s = _sincos_2pi(d)
        fc_re = jnp.sum(c, axis=0, keepdims=True)
        fc_im = jnp.sum(s, axis=0, keepdims=True)
        fm = jnp.sqrt(fc_re * fc_re + fc_im * fc_im + 1e-12)
        fm_scr[i:i + 1, :] = fm

        valid_row = valid_ref[i:i + 1, :]
        vals = fm * fm / eps_ref[i:i + 1, :] * valid_row
        m = bins2d == lab_ref[i:i + 1, :]
        seg = seg + jnp.where(m, vals, 0.0)
        cnt = cnt + jnp.where(m, valid_row, 0.0)

    seg_t = jnp.sum(seg, axis=1, keepdims=True)
    cnt_t = jnp.sum(cnt, axis=1, keepdims=True)
    sigma_p = seg_t / jnp.maximum(cnt_t, 1.0)
    sa_clip = jnp.clip(sa_ref[...], 0.015, 0.99)

    labels = lab_ref[...]
    sp_g = jnp.zeros((NB, R), jnp.float32)
    sa_g = jnp.zeros((NB, R), jnp.float32)
    for b in range(N_BINS):
        mb = labels == b
        sp_g = jnp.where(mb, sigma_p[b, 0], sp_g)
        sa_g = jnp.where(mb, sa_clip[b, 0], sa_g)

    fm_all = fm_scr[...]
    eobs = eobs_ref[...]
    dobs = dobs_ref[...]
    eps = eps_ref[...]

    ecalc = fm_all / jnp.sqrt(eps * sp_g)
    dsa = dobs * sa_g
    t = 1.0 - dsa * dsa
    e2 = eobs * eobs + ecalc * ecalc
    arg_a = 2.0 * dsa * eobs * ecalc / t
    llg_a = -jnp.log(t) - dsa * dsa * e2 / t + _log_i0(arg_a)
    arg_c = jnp.abs(0.5 * arg_a)
    logcosh = arg_c + jnp.log(1.0 + jnp.exp(-2.0 * arg_c)) - jnp.float32(jnp.log(2.0))
    llg_c = -0.5 * jnp.log(t) - dsa * dsa * e2 / (2.0 * t) + logcosh
    llg = jnp.where(cent_ref[...] != 0.0, llg_c, llg_a)
    out_ref[...] = jnp.sum(llg * sub_ref[...]).reshape(1, 1)


@functools.partial(jax.jit, static_argnames=())
def kernel(xyz_ort, Eobs, Eps, Dobs, sigmaAs, Centric, bin_labels, hkl, sub_mask):
    f32 = jnp.float32
    frac = (xyz_ort / CELL).astype(f32)

    def pad1(x, fill):
        return jnp.pad(x.astype(f32), (0, NP - N_HKL), constant_values=fill)

    hkl_t = jnp.pad(hkl.astype(f32).T, ((0, 0), (0, NP - N_HKL)))
    eobs = pad1(Eobs, 0.0).reshape(NB, R)
    eps = pad1(Eps, 1.0).reshape(NB, R)
    dobs = pad1(Dobs, 0.0).reshape(NB, R)
    cent = pad1(Centric.astype(f32), 0.0).reshape(NB, R)
    sub = pad1(sub_mask.astype(f32), 0.0).reshape(NB, R)
    valid = pad1(jnp.ones((N_HKL,), f32), 0.0).reshape(NB, R)
    labels = jnp.pad(bin_labels, (0, NP - N_HKL)).reshape(NB, R)
    sa = jnp.pad(sigmaAs.astype(f32), (0, BINS_PAD - N_BINS)).reshape(BINS_PAD, 1)

    out = pl.pallas_call(
        _llg_kernel,
        out_shape=jax.ShapeDtypeStruct((1, 1), f32),
        scratch_shapes=[
            pltpu.VMEM((NB, R), f32),
        ],
    )(frac, hkl_t, eobs, eps, dobs, cent, labels, sub, valid, sa)
    return out[0, 0]


# speedup vs baseline: 1.3539x; 1.0006x over previous
import functools

import jax
import jax.numpy as jnp
from jax.experimental import pallas as pl
from jax.experimental.pallas import tpu as pltpu

N_ATOMS = 256
N_HKL = 50000
N_BINS = 20
CELL = 50.0

NP = 51200
NB = 16
R = NP // NB
BINS_PAD = 32


def _log_i0(x):
    ax = jnp.abs(x)
    t = jnp.square(jnp.minimum(ax, 3.75) / 3.75)
    p_small = 1.0 + t * (3.5156229 + t * (3.0899424 + t * (1.2067492
        + t * (0.2659732 + t * (0.0360768 + t * 0.0045813)))))
    small = jnp.log(p_small)
    xl = jnp.maximum(ax, 3.75)
    u = 3.75 / xl
    p_large = 0.39894228 + u * (0.01328592 + u * (0.00225319 + u * (-0.00157565
        + u * (0.00916281 + u * (-0.02057706 + u * (0.02635537
        + u * (-0.01647633 + u * 0.00392377)))))))
    large = xl - 0.5 * jnp.log(xl) + jnp.log(p_large)
    return jnp.where(ax <= 3.75, small, large)


def _sincos_2pi(d):
    u = d - jnp.round(d)
    z = u * u
    c = 0.9999814 + z * (-19.73259 + z * (64.69856 + z * (
        -82.54686 + z * 45.912495)))
    s = u * (6.2831745 + z * (-41.337803 + z * (81.46372 + z * (
        -75.001564 + z * 33.720444))))
    return c, s


def _colsum(m):
    while m.shape[0] > 8:
        h = m.shape[0] // 2
        m = m[:h] + m[h:]
    return jnp.sum(m, axis=0, keepdims=True)


def _llg_kernel(frac_ref, hklt_ref, eobs_ref, eps_ref, dobs_ref, cent_ref,
                lab_ref, sub_ref, valid_ref, sa_ref, out_ref, fm_scr):
    frac = frac_ref[...]

    bins2d = jax.lax.broadcasted_iota(jnp.int32, (BINS_PAD, R), 0)

    seg = jnp.zeros((BINS_PAD, R), jnp.float32)
    cnt = jnp.zeros((BINS_PAD, R), jnp.float32)

    for i in range(NB):
        hkl_blk = hklt_ref[:, i * R:(i + 1) * R]
        d = jax.lax.dot_general(
            frac, hkl_blk, (((1,), (0,)), ((), ())),
            preferred_element_type=jnp.float32)
        c, s = _sincos_2pi(d)
        fc_re = _colsum(c)
        fc_im = _colsum(s)
        fm = jnp.sqrt(fc_re * fc_re + fc_im * fc_im + 1e-12)
        fm_scr[i:i + 1, :] = fm

        valid_row = valid_ref[i:i + 1, :]
        vals = fm * fm / eps_ref[i:i + 1, :] * valid_row
        m = bins2d == lab_ref[i:i + 1, :]
        seg = seg + jnp.where(m, vals, 0.0)
        cnt = cnt + jnp.where(m, valid_row, 0.0)

    seg_t = jnp.sum(seg, axis=1, keepdims=True)
    cnt_t = jnp.sum(cnt, axis=1, keepdims=True)
    sigma_p = seg_t / jnp.maximum(cnt_t, 1.0)
    sa_clip = jnp.clip(sa_ref[...], 0.015, 0.99)

    labels = lab_ref[...]
    sp_g = jnp.zeros((NB, R), jnp.float32)
    sa_g = jnp.zeros((NB, R), jnp.float32)
    for b in range(N_BINS):
        mb = labels == b
        sp_g = jnp.where(mb, sigma_p[b, 0], sp_g)
        sa_g = jnp.where(mb, sa_clip[b, 0], sa_g)

    fm_all = fm_scr[...]
    eobs = eobs_ref[...]
    dobs = dobs_ref[...]
    eps = eps_ref[...]

    ecalc = fm_all / jnp.sqrt(eps * sp_g)
    dsa = dobs * sa_g
    t = 1.0 - dsa * dsa
    e2 = eobs * eobs + ecalc * ecalc
    arg_a = 2.0 * dsa * eobs * ecalc / t
    llg_a = -jnp.log(t) - dsa * dsa * e2 / t + _log_i0(arg_a)
    arg_c = jnp.abs(0.5 * arg_a)
    logcosh = arg_c + jnp.log(1.0 + jnp.exp(-2.0 * arg_c)) - jnp.float32(jnp.log(2.0))
    llg_c = -0.5 * jnp.log(t) - dsa * dsa * e2 / (2.0 * t) + logcosh
    llg = jnp.where(cent_ref[...] != 0.0, llg_c, llg_a)
    out_ref[...] = jnp.sum(llg * sub_ref[...]).reshape(1, 1)


@functools.partial(jax.jit, static_argnames=())
def kernel(xyz_ort, Eobs, Eps, Dobs, sigmaAs, Centric, bin_labels, hkl, sub_mask):
    f32 = jnp.float32
    frac = (xyz_ort / CELL).astype(f32)

    def pad1(x, fill):
        return jnp.pad(x.astype(f32), (0, NP - N_HKL), constant_values=fill)

    hkl_t = jnp.pad(hkl.astype(f32).T, ((0, 0), (0, NP - N_HKL)))
    eobs = pad1(Eobs, 0.0).reshape(NB, R)
    eps = pad1(Eps, 1.0).reshape(NB, R)
    dobs = pad1(Dobs, 0.0).reshape(NB, R)
    cent = pad1(Centric.astype(f32), 0.0).reshape(NB, R)
    sub = pad1(sub_mask.astype(f32), 0.0).reshape(NB, R)
    valid = pad1(jnp.ones((N_HKL,), f32), 0.0).reshape(NB, R)
    labels = jnp.pad(bin_labels, (0, NP - N_HKL)).reshape(NB, R)
    sa = jnp.pad(sigmaAs.astype(f32), (0, BINS_PAD - N_BINS)).reshape(BINS_PAD, 1)

    out = pl.pallas_call(
        _llg_kernel,
        out_shape=jax.ShapeDtypeStruct((1, 1), f32),
        scratch_shapes=[
            pltpu.VMEM((NB, R), f32),
        ],
    )(frac, hkl_t, eobs, eps, dobs, cent, labels, sub, valid, sa)
    return out[0, 0]


# speedup vs baseline: 1.4410x; 1.0643x over previous
import functools

import jax
import jax.numpy as jnp
from jax.experimental import pallas as pl
from jax.experimental.pallas import tpu as pltpu

N_ATOMS = 256
N_HKL = 50000
N_BINS = 20
CELL = 50.0

NP = 51200
NB = 16
R = NP // NB
BINS_PAD = 32


def _log_i0(x):
    ax = jnp.abs(x)
    t = jnp.square(jnp.minimum(ax, 3.75) / 3.75)
    p_small = 1.0 + t * (3.5156229 + t * (3.0899424 + t * (1.2067492
        + t * (0.2659732 + t * (0.0360768 + t * 0.0045813)))))
    small = jnp.log(p_small)
    xl = jnp.maximum(ax, 3.75)
    u = 3.75 / xl
    p_large = 0.39894228 + u * (0.01328592 + u * (0.00225319 + u * (-0.00157565
        + u * (0.00916281 + u * (-0.02057706 + u * (0.02635537
        + u * (-0.01647633 + u * 0.00392377)))))))
    large = xl - 0.5 * jnp.log(xl) + jnp.log(p_large)
    return jnp.where(ax <= 3.75, small, large)


def _sincos_2pi(d):
    u = d - jnp.round(d)
    z = u * u
    c = 0.9999814 + z * (-19.73259 + z * (64.69856 + z * (
        -82.54686 + z * 45.912495)))
    s = u * (6.282679 + z * (-41.22836 + z * (79.02798 + z * -58.6237)))
    return c, s


def _llg_kernel(frac_ref, hklt_ref, eobs_ref, eps_ref, dobs_ref, cent_ref,
                lab_ref, sub_ref, valid_ref, sa_ref, out_ref, fm_scr):
    frac = frac_ref[...]

    bins2d = jax.lax.broadcasted_iota(jnp.int32, (BINS_PAD, R), 0)

    seg = jnp.zeros((BINS_PAD, R), jnp.float32)
    cnt = jnp.zeros((BINS_PAD, R), jnp.float32)

    for i in range(NB):
        hkl_blk = hklt_ref[:, i * R:(i + 1) * R]
        d = jax.lax.dot_general(
            frac, hkl_blk, (((1,), (0,)), ((), ())),
            preferred_element_type=jnp.float32)
        c, s = _sincos_2pi(d)
        fc_re = jnp.sum(c, axis=0, keepdims=True)
        fc_im = jnp.sum(s, axis=0, keepdims=True)
        fm = jnp.sqrt(fc_re * fc_re + fc_im * fc_im + 1e-12)
        fm_scr[i:i + 1, :] = fm

        valid_row = valid_ref[i:i + 1, :]
        vals = fm * fm / eps_ref[i:i + 1, :] * valid_row
        m = bins2d == lab_ref[i:i + 1, :]
        seg = seg + jnp.where(m, vals, 0.0)
        cnt = cnt + jnp.where(m, valid_row, 0.0)

    seg_t = jnp.sum(seg, axis=1, keepdims=True)
    cnt_t = jnp.sum(cnt, axis=1, keepdims=True)
    sigma_p = seg_t / jnp.maximum(cnt_t, 1.0)
    sa_clip = jnp.clip(sa_ref[...], 0.015, 0.99)

    labels = lab_ref[...]
    sp_g = jnp.zeros((NB, R), jnp.float32)
    sa_g = jnp.zeros((NB, R), jnp.float32)
    for b in range(N_BINS):
        mb = labels == b
        sp_g = jnp.where(mb, sigma_p[b, 0], sp_g)
        sa_g = jnp.where(mb, sa_clip[b, 0], sa_g)

    fm_all = fm_scr[...]
    eobs = eobs_ref[...]
    dobs = dobs_ref[...]
    eps = eps_ref[...]

    ecalc = fm_all / jnp.sqrt(eps * sp_g)
    dsa = dobs * sa_g
    t = 1.0 - dsa * dsa
    e2 = eobs * eobs + ecalc * ecalc
    arg_a = 2.0 * dsa * eobs * ecalc / t
    llg_a = -jnp.log(t) - dsa * dsa * e2 / t + _log_i0(arg_a)
    arg_c = jnp.abs(0.5 * arg_a)
    logcosh = arg_c + jnp.log(1.0 + jnp.exp(-2.0 * arg_c)) - jnp.float32(jnp.log(2.0))
    llg_c = -0.5 * jnp.log(t) - dsa * dsa * e2 / (2.0 * t) + logcosh
    llg = jnp.where(cent_ref[...] != 0.0, llg_c, llg_a)
    out_ref[...] = jnp.sum(llg * sub_ref[...]).reshape(1, 1)


@functools.partial(jax.jit, static_argnames=())
def kernel(xyz_ort, Eobs, Eps, Dobs, sigmaAs, Centric, bin_labels, hkl, sub_mask):
    f32 = jnp.float32
    frac = (xyz_ort / CELL).astype(f32)

    def pad1(x, fill):
        return jnp.pad(x.astype(f32), (0, NP - N_HKL), constant_values=fill)

    hkl_t = jnp.pad(hkl.astype(f32).T, ((0, 0), (0, NP - N_HKL)))
    eobs = pad1(Eobs, 0.0).reshape(NB, R)
    eps = pad1(Eps, 1.0).reshape(NB, R)
    dobs = pad1(Dobs, 0.0).reshape(NB, R)
    cent = pad1(Centric.astype(f32), 0.0).reshape(NB, R)
    sub = pad1(sub_mask.astype(f32), 0.0).reshape(NB, R)
    valid = pad1(jnp.ones((N_HKL,), f32), 0.0).reshape(NB, R)
    labels = jnp.pad(bin_labels, (0, NP - N_HKL)).reshape(NB, R)
    sa = jnp.pad(sigmaAs.astype(f32), (0, BINS_PAD - N_BINS)).reshape(BINS_PAD, 1)

    out = pl.pallas_call(
        _llg_kernel,
        out_shape=jax.ShapeDtypeStruct((1, 1), f32),
        scratch_shapes=[
            pltpu.VMEM((NB, R), f32),
        ],
    )(frac, hkl_t, eobs, eps, dobs, cent, labels, sub, valid, sa)
    return out[0, 0]
